# baseline (speedup 1.0000x reference)
import functools

import jax
import jax.numpy as jnp
from jax import lax
from jax.experimental import pallas as pl
from jax.experimental.pallas import tpu as pltpu
from jax.experimental.pallas import tpu_sc as plsc

_BATCH = 16384
_D = 100
_HALF = _D // 2
_NW = 32
_ROWS = _BATCH // _NW
_UNROLL = 8

_COLS = (0, 16, 32, 48, 64, 80, 84)


def _body(x_hbm, out_hbm, in_v, out_v):
    wid = lax.axis_index("s") * 2 + lax.axis_index("c")
    base = wid * _ROWS

    pltpu.sync_copy(x_hbm.at[pl.ds(base, _ROWS)], in_v)

    lane = lax.iota(jnp.int32, 16)
    src = []
    for c0 in _COLS:
        j = c0 + lane
        src.append(jnp.where(j < _HALF, 2 * j, 2 * (j - _HALF) + 1))

    def block(i, carry):
        r0 = i * _UNROLL
        for rr in range(_UNROLL):
            r = r0 + rr
            rv = jnp.broadcast_to(r, (16,))
            for jj, c0 in enumerate(_COLS):
                v = plsc.load_gather(in_v, [rv, src[jj]])
                out_v[r, pl.ds(c0, 16)] = v
        return carry

    lax.fori_loop(0, _ROWS // _UNROLL, block, 0)

    pltpu.sync_copy(out_v, out_hbm.at[pl.ds(base, _ROWS)])


_sc_deinterleave = functools.partial(
    pl.kernel,
    mesh=plsc.VectorSubcoreMesh(core_axis_name="c", subcore_axis_name="s"),
    out_type=jax.ShapeDtypeStruct((_BATCH, _D), jnp.float32),
    scratch_types=[
        pltpu.VMEM((_ROWS, _D), jnp.float32),
        pltpu.VMEM((_ROWS, _D), jnp.float32),
    ],
    compiler_params=pltpu.CompilerParams(
        needs_layout_passes=False, disable_bounds_checks=True),
)(_body)


def kernel(x):
    return _sc_deinterleave(x).reshape(_BATCH, _D, 1)


# speedup vs baseline: 2.0110x; 2.0110x over previous
import functools

import jax
import jax.numpy as jnp
from jax import lax
from jax.experimental import pallas as pl
from jax.experimental.pallas import tpu as pltpu
from jax.experimental.pallas import tpu_sc as plsc

_BATCH = 16384
_D = 100
_HALF = _D // 2
_NW = 32
_MAXF = 4


def _body(xt_hbm, out_hbm, stage):
    wid = lax.axis_index("s") * 2 + lax.axis_index("c")
    nf = jnp.where(wid < 4, 4, 3)
    j0 = jnp.where(wid < 4, 4 * wid, 16 + 3 * (wid - 4))

    def do_feature(i, carry):
        j = j0 + i
        c = jnp.where(j < _HALF, 2 * j, 2 * j - (_D - 1))
        pltpu.sync_copy(xt_hbm.at[c], stage.at[i])
        pltpu.sync_copy(stage.at[i], out_hbm.at[j, 0])
        return carry

    lax.fori_loop(0, nf, do_feature, 0)


_sc_permute_rows = functools.partial(
    pl.kernel,
    mesh=plsc.VectorSubcoreMesh(core_axis_name="c", subcore_axis_name="s"),
    out_type=jax.ShapeDtypeStruct((_D, 1, _BATCH), jnp.float32),
    scratch_types=[
        pltpu.VMEM((_MAXF, _BATCH), jnp.float32),
    ],
    compiler_params=pltpu.CompilerParams(
        needs_layout_passes=False, disable_bounds_checks=True),
)(_body)


def kernel(x):
    yt = _sc_permute_rows(x.T)
    return jnp.transpose(yt, (2, 0, 1))
